# Initial kernel scaffold; baseline (speedup 1.0000x reference)
import jax
import jax.numpy as jnp
from jax.experimental import pallas as pl


def kernel(x, edge_index, kernel, bias):
    raise NotImplementedError("write your pallas kernel here")


# speedup vs baseline: 6.1748x; 6.1748x over previous
import functools

import jax
import jax.numpy as jnp
from jax import lax
from jax.experimental import pallas as pl
from jax.experimental.pallas import tpu as pltpu
from jax.experimental.pallas import tpu_sc as plsc

_N = 10000
_E = 320000
_D = 128

_NC = 2
_NS = 16
_NW = _NC * _NS
_EPW = _E // _NW
_C = 128
_NFULL = _EPW // _C
_TAIL = _EPW - _NFULL * _C
_RPT = _N // _NS
_CPR = 624
_ZR = 125
_DEGW = 16


def _sc_agg_body(x_hbm, src_hbm, dst_hbm, agg_out, deg_out,
                 idx_s, idx_d, rows, idx_st, idx_dt, rows_t,
                 ones_v, ones_t, zbuf, zbuf_d, agg_sh, deg_sh, sem):
    cid = lax.axis_index("c")
    sid = lax.axis_index("s")
    wid = sid * _NC + cid
    ebase = wid * _EPW

    zeros16 = jnp.zeros((16,), jnp.float32)
    ones16 = jnp.ones((16,), jnp.float32)

    def fill_z(r, carry):
        for j in range(_D // 16):
            zbuf[r, pl.ds(j * 16, 16)] = zeros16
        zbuf_d[r, :] = zeros16
        return carry

    lax.fori_loop(0, _ZR, fill_z, 0)

    def fill_o(r, carry):
        ones_v[r, :] = ones16
        return carry

    lax.fori_loop(0, _C, fill_o, 0)

    def fill_ot(r, carry):
        ones_t[r, :] = ones16
        return carry

    lax.fori_loop(0, _TAIL, fill_ot, 0)

    for t in range(_RPT // _ZR):
        rb = sid * _RPT + t * _ZR
        pltpu.sync_copy(zbuf, agg_sh.at[pl.ds(rb, _ZR)])
        pltpu.sync_copy(zbuf_d, deg_sh.at[pl.ds(rb, _ZR)])
    plsc.subcore_barrier()

    def chunk(g, carry):
        off = pl.multiple_of(ebase + g * _C, 8)
        pltpu.sync_copy(src_hbm.at[pl.ds(off, _C)], idx_s)
        pltpu.sync_copy(dst_hbm.at[pl.ds(off, _C)], idx_d)
        pltpu.async_copy(x_hbm.at[idx_s], rows, sem).wait()
        pltpu.sync_copy(rows, agg_sh.at[idx_d], add=True)
        pltpu.sync_copy(ones_v, deg_sh.at[idx_s], add=True)
        return carry

    lax.fori_loop(0, _NFULL, chunk, 0)

    offt = pl.multiple_of(ebase + _NFULL * _C, 8)
    pltpu.sync_copy(src_hbm.at[pl.ds(offt, _TAIL)], idx_st)
    pltpu.sync_copy(dst_hbm.at[pl.ds(offt, _TAIL)], idx_dt)
    pltpu.async_copy(x_hbm.at[idx_st], rows_t, sem).wait()
    pltpu.sync_copy(rows_t, agg_sh.at[idx_dt], add=True)
    pltpu.sync_copy(ones_t, deg_sh.at[idx_st], add=True)

    plsc.subcore_barrier()

    rb = pl.multiple_of(sid * _CPR, 8)
    pltpu.sync_copy(agg_sh.at[pl.ds(rb, _CPR)], agg_out.at[cid, pl.ds(rb, _CPR)])
    pltpu.sync_copy(deg_sh.at[pl.ds(rb, _CPR)], deg_out.at[cid, pl.ds(rb, _CPR)])

    @pl.when(sid == 0)
    def _copy_tail():
        tb = _NS * _CPR
        pltpu.sync_copy(agg_sh.at[pl.ds(tb, _N - tb)],
                        agg_out.at[cid, pl.ds(tb, _N - tb)])
        pltpu.sync_copy(deg_sh.at[pl.ds(tb, _N - tb)],
                        deg_out.at[cid, pl.ds(tb, _N - tb)])


_sc_agg = functools.partial(
    pl.kernel,
    mesh=plsc.VectorSubcoreMesh(core_axis_name="c", subcore_axis_name="s"),
    out_type=[
        jax.ShapeDtypeStruct((_NC, _N, _D), jnp.float32),
        jax.ShapeDtypeStruct((_NC, _N, _DEGW), jnp.float32),
    ],
    scratch_types=[
        pltpu.VMEM((_C,), jnp.int32),
        pltpu.VMEM((_C,), jnp.int32),
        pltpu.VMEM((_C, _D), jnp.float32),
        pltpu.VMEM((_TAIL,), jnp.int32),
        pltpu.VMEM((_TAIL,), jnp.int32),
        pltpu.VMEM((_TAIL, _D), jnp.float32),
        pltpu.VMEM((_C, _DEGW), jnp.float32),
        pltpu.VMEM((_TAIL, _DEGW), jnp.float32),
        pltpu.VMEM((_ZR, _D), jnp.float32),
        pltpu.VMEM((_ZR, _DEGW), jnp.float32),
        pltpu.VMEM_SHARED((_N, _D), jnp.float32),
        pltpu.VMEM_SHARED((_N, _DEGW), jnp.float32),
        pltpu.SemaphoreType.DMA,
    ],
    compiler_params=pltpu.CompilerParams(use_tc_tiling_on_sc=False),
)(_sc_agg_body)


_BN = 1000


def _tc_body(p0, p1, d0, d1, w, b, o):
    deg = d0[:, 0:1] + d1[:, 0:1]
    norm = lax.rsqrt(deg)
    h = jnp.dot(p0[...] + p1[...], w[...], preferred_element_type=jnp.float32)
    o[...] = jnp.maximum(h * norm + b[...], 0.0)


def _tc_finish(p0, p1, d0, d1, w, b2d):
    return pl.pallas_call(
        _tc_body,
        grid=(_N // _BN,),
        in_specs=[
            pl.BlockSpec((_BN, _D), lambda i: (i, 0)),
            pl.BlockSpec((_BN, _D), lambda i: (i, 0)),
            pl.BlockSpec((_BN, _DEGW), lambda i: (i, 0)),
            pl.BlockSpec((_BN, _DEGW), lambda i: (i, 0)),
            pl.BlockSpec((_D, _D), lambda i: (0, 0)),
            pl.BlockSpec((1, _D), lambda i: (0, 0)),
        ],
        out_specs=pl.BlockSpec((_BN, _D), lambda i: (i, 0)),
        out_shape=jax.ShapeDtypeStruct((_N, _D), jnp.float32),
    )(p0, p1, d0, d1, w, b2d)


def kernel(x, edge_index, kernel, bias):
    src = edge_index[0]
    dst = edge_index[1]
    agg, deg = _sc_agg(x, src, dst)
    return _tc_finish(agg[0], agg[1], deg[0], deg[1], kernel,
                      bias.reshape(1, _D))


# speedup vs baseline: 8.8832x; 1.4386x over previous
import functools

import jax
import jax.numpy as jnp
from jax import lax
from jax.experimental import pallas as pl
from jax.experimental.pallas import tpu as pltpu
from jax.experimental.pallas import tpu_sc as plsc

_N = 10000
_E = 320000
_D = 128

_NC = 2
_NS = 16
_NW = _NC * _NS
_EPW = _E // _NW
_C = 128
_NFULL = _EPW // _C
_TAIL = _EPW - _NFULL * _C
_RPT = _N // _NS
_CPR = 624
_ZR = 125
_DEGW = 16


def _sc_agg_body(x_hbm, src_hbm, dst_hbm, agg_out, deg_out,
                 idx_s2, idx_d2, rows, idx_st, idx_dt, rows_t,
                 ones_v, zbuf_d, agg_sh, deg_sh,
                 gsem0, gsem1, semt):
    gsems = (gsem0, gsem1)
    cid = lax.axis_index("c")
    sid = lax.axis_index("s")
    wid = sid * _NC + cid
    ebase = wid * _EPW

    zeros16 = jnp.zeros((16,), jnp.float32)
    ones16 = jnp.ones((16,), jnp.float32)

    def fill_z(r, carry):
        for j in range(_D // 16):
            rows[0, r, pl.ds(j * 16, 16)] = zeros16
        zbuf_d[r, :] = zeros16
        return carry

    lax.fori_loop(0, _ZR, fill_z, 0)

    def fill_o(r, carry):
        ones_v[r, :] = ones16
        return carry

    lax.fori_loop(0, _C, fill_o, 0)

    zrows = rows.at[0, pl.ds(0, _ZR)]
    for t in range(_RPT // _ZR):
        rb = sid * _RPT + t * _ZR
        pltpu.sync_copy(zrows, agg_sh.at[pl.ds(rb, _ZR)])
        pltpu.sync_copy(zbuf_d, deg_sh.at[pl.ds(rb, _ZR)])
    plsc.subcore_barrier()

    def load_idx(g, b):
        off = pl.multiple_of(ebase + g * _C, 8)
        pltpu.sync_copy(src_hbm.at[pl.ds(off, _C)], idx_s2.at[b])
        pltpu.sync_copy(dst_hbm.at[pl.ds(off, _C)], idx_d2.at[b])

    def _gather_copy(b):
        return pltpu.make_async_copy(
            x_hbm.at[idx_s2.at[b]], rows.at[b], gsems[b])

    load_idx(0, 0)
    _gather_copy(0).start()
    load_idx(1, 1)
    _gather_copy(1).start()

    def body(t, carry):
        for j in range(2):
            g = 2 * t + j
            _gather_copy(j).wait()
            pltpu.sync_copy(rows.at[j], agg_sh.at[idx_d2.at[j]], add=True)
            pltpu.sync_copy(ones_v, deg_sh.at[idx_s2.at[j]], add=True)

            @pl.when(g + 2 < _NFULL)
            def _():
                load_idx(g + 2, j)
                _gather_copy(j).start()
        return carry

    lax.fori_loop(0, _NFULL // 2, body, 0)

    offt = pl.multiple_of(ebase + _NFULL * _C, 8)
    pltpu.sync_copy(src_hbm.at[pl.ds(offt, _TAIL)], idx_st)
    pltpu.sync_copy(dst_hbm.at[pl.ds(offt, _TAIL)], idx_dt)
    pltpu.async_copy(x_hbm.at[idx_st], rows_t, semt).wait()
    pltpu.sync_copy(rows_t, agg_sh.at[idx_dt], add=True)
    pltpu.sync_copy(ones_v.at[pl.ds(0, _TAIL)], deg_sh.at[idx_st], add=True)

    plsc.subcore_barrier()

    rb = pl.multiple_of(sid * _CPR, 8)
    pltpu.sync_copy(agg_sh.at[pl.ds(rb, _CPR)], agg_out.at[cid, pl.ds(rb, _CPR)])
    pltpu.sync_copy(deg_sh.at[pl.ds(rb, _CPR)], deg_out.at[cid, pl.ds(rb, _CPR)])

    @pl.when(sid == 0)
    def _copy_tail():
        tb = _NS * _CPR
        pltpu.sync_copy(agg_sh.at[pl.ds(tb, _N - tb)],
                        agg_out.at[cid, pl.ds(tb, _N - tb)])
        pltpu.sync_copy(deg_sh.at[pl.ds(tb, _N - tb)],
                        deg_out.at[cid, pl.ds(tb, _N - tb)])


_sc_agg = functools.partial(
    pl.kernel,
    mesh=plsc.VectorSubcoreMesh(core_axis_name="c", subcore_axis_name="s"),
    out_type=[
        jax.ShapeDtypeStruct((_NC, _N, _D), jnp.float32),
        jax.ShapeDtypeStruct((_NC, _N, _DEGW), jnp.float32),
    ],
    scratch_types=[
        pltpu.VMEM((2, _C), jnp.int32),
        pltpu.VMEM((2, _C), jnp.int32),
        pltpu.VMEM((2, _C, _D), jnp.float32),
        pltpu.VMEM((_TAIL,), jnp.int32),
        pltpu.VMEM((_TAIL,), jnp.int32),
        pltpu.VMEM((_TAIL, _D), jnp.float32),
        pltpu.VMEM((_C, _DEGW), jnp.float32),
        pltpu.VMEM((_ZR, _DEGW), jnp.float32),
        pltpu.VMEM_SHARED((_N, _D), jnp.float32),
        pltpu.VMEM_SHARED((_N, _DEGW), jnp.float32),
        pltpu.SemaphoreType.DMA,
        pltpu.SemaphoreType.DMA,
        pltpu.SemaphoreType.DMA,
    ],
    compiler_params=pltpu.CompilerParams(use_tc_tiling_on_sc=False),
)(_sc_agg_body)


_BN = 1000


def _tc_body(p0, p1, d0, d1, w, b, o):
    deg = d0[:, 0:1] + d1[:, 0:1]
    norm = lax.rsqrt(deg)
    h = jnp.dot(p0[...] + p1[...], w[...], preferred_element_type=jnp.float32)
    o[...] = jnp.maximum(h * norm + b[...], 0.0)


def _tc_finish(p0, p1, d0, d1, w, b2d):
    return pl.pallas_call(
        _tc_body,
        grid=(_N // _BN,),
        in_specs=[
            pl.BlockSpec((_BN, _D), lambda i: (i, 0)),
            pl.BlockSpec((_BN, _D), lambda i: (i, 0)),
            pl.BlockSpec((_BN, _DEGW), lambda i: (i, 0)),
            pl.BlockSpec((_BN, _DEGW), lambda i: (i, 0)),
            pl.BlockSpec((_D, _D), lambda i: (0, 0)),
            pl.BlockSpec((1, _D), lambda i: (0, 0)),
        ],
        out_specs=pl.BlockSpec((_BN, _D), lambda i: (i, 0)),
        out_shape=jax.ShapeDtypeStruct((_N, _D), jnp.float32),
    )(p0, p1, d0, d1, w, b2d)


def kernel(x, edge_index, kernel, bias):
    src = edge_index[0]
    dst = edge_index[1]
    agg, deg = _sc_agg(x, src, dst)
    return _tc_finish(agg[0], agg[1], deg[0], deg[1], kernel,
                      bias.reshape(1, _D))


# speedup vs baseline: 12.0566x; 1.3572x over previous
import functools

import jax
import jax.numpy as jnp
from jax import lax
from jax.experimental import pallas as pl
from jax.experimental.pallas import tpu as pltpu
from jax.experimental.pallas import tpu_sc as plsc

_N = 10000
_E = 320000
_D = 128

_NC = 2
_NS = 16
_NW = _NC * _NS
_EPW = _E // _NW
_C = 128
_NFULL = _EPW // _C
_TAIL = _EPW - _NFULL * _C
_RPT = _N // _NS
_CPR = 624
_ZR = 125
_DEGW = 16
_NI = 4
_UN = 4
_NLOOP = _NFULL // _UN
_EPI = _NFULL - _NLOOP * _UN


def _sc_agg_body(x_hbm, src_hbm, dst_hbm,
                 agg_out0, agg_out1, deg_out0, deg_out1,
                 idx_s4, idx_d4, rows, idx_st, idx_dt, rows_t,
                 ones_v, zbuf_d, agg_sh, deg_sh,
                 isem0, isem1, isem2, isem3, gsem0, gsem1, semt):
    isems = (isem0, isem1, isem2, isem3)
    gsems = (gsem0, gsem1)
    cid = lax.axis_index("c")
    sid = lax.axis_index("s")
    wid = sid * _NC + cid
    ebase = wid * _EPW

    zeros16 = jnp.zeros((16,), jnp.float32)
    ones16 = jnp.ones((16,), jnp.float32)

    def fill_z(r, carry):
        for j in range(_D // 16):
            rows[0, r, pl.ds(j * 16, 16)] = zeros16
        zbuf_d[r, :] = zeros16
        return carry

    lax.fori_loop(0, _ZR, fill_z, 0)

    def fill_o(r, carry):
        ones_v[r, :] = ones16
        return carry

    lax.fori_loop(0, _C, fill_o, 0)

    zrows = rows.at[0, pl.ds(0, _ZR)]
    for t in range(_RPT // _ZR):
        rb = sid * _RPT + t * _ZR
        pltpu.sync_copy(zrows, agg_sh.at[pl.ds(rb, _ZR)])
        pltpu.sync_copy(zbuf_d, deg_sh.at[pl.ds(rb, _ZR)])
    plsc.subcore_barrier()

    def _idx_copies(g, r):
        off = pl.multiple_of(ebase + g * _C, 8)
        return (
            pltpu.make_async_copy(src_hbm.at[pl.ds(off, _C)], idx_s4.at[r],
                                  isems[r]),
            pltpu.make_async_copy(dst_hbm.at[pl.ds(off, _C)], idx_d4.at[r],
                                  isems[r]),
        )

    def issue_idx(g, r):
        for c in _idx_copies(g, r):
            c.start()

    def wait_idx(g, r):
        for c in _idx_copies(g, r):
            c.wait()

    def _gather_copy(r, b):
        return pltpu.make_async_copy(
            x_hbm.at[idx_s4.at[r]], rows.at[b], gsems[b])

    for g in range(_NI):
        issue_idx(g, g)
    wait_idx(0, 0)
    _gather_copy(0, 0).start()
    wait_idx(1, 1)
    _gather_copy(1, 1).start()

    def chunk_step(g, j, r, guard):
        _gather_copy(r, j).wait()
        pltpu.sync_copy(rows.at[j], agg_sh.at[idx_d4.at[r]], add=True)
        pltpu.sync_copy(ones_v, deg_sh.at[idx_s4.at[r]], add=True)
        if guard:
            @pl.when(g + _NI < _NFULL)
            def _():
                issue_idx(g + _NI, r)

            @pl.when(g + 2 < _NFULL)
            def _():
                wait_idx(g + 2, (r + 2) % _NI)
                _gather_copy((r + 2) % _NI, j).start()
        else:
            if g + _NI < _NFULL:
                issue_idx(g + _NI, r)
            if g + 2 < _NFULL:
                wait_idx(g + 2, (g + 2) % _NI)
                _gather_copy((g + 2) % _NI, j).start()

    def body(t, carry):
        for u in range(_UN):
            g = _UN * t + u
            chunk_step(g, u % 2, u % _NI, True)
        return carry

    lax.fori_loop(0, _NLOOP, body, 0)

    for g in range(_NLOOP * _UN, _NFULL):
        chunk_step(g, g % 2, g % _NI, False)

    offt = pl.multiple_of(ebase + _NFULL * _C, 8)
    pltpu.sync_copy(src_hbm.at[pl.ds(offt, _TAIL)], idx_st)
    pltpu.sync_copy(dst_hbm.at[pl.ds(offt, _TAIL)], idx_dt)
    pltpu.async_copy(x_hbm.at[idx_st], rows_t, semt).wait()
    pltpu.sync_copy(rows_t, agg_sh.at[idx_dt], add=True)
    pltpu.sync_copy(ones_v.at[pl.ds(0, _TAIL)], deg_sh.at[idx_st], add=True)

    plsc.subcore_barrier()

    rb = pl.multiple_of(sid * _CPR, 8)
    tb = _NS * _CPR

    @pl.when(cid == 0)
    def _out0():
        pltpu.sync_copy(agg_sh.at[pl.ds(rb, _CPR)], agg_out0.at[pl.ds(rb, _CPR)])
        pltpu.sync_copy(deg_sh.at[pl.ds(rb, _CPR)], deg_out0.at[pl.ds(rb, _CPR)])

        @pl.when(sid == 0)
        def _tail0():
            pltpu.sync_copy(agg_sh.at[pl.ds(tb, _N - tb)],
                            agg_out0.at[pl.ds(tb, _N - tb)])
            pltpu.sync_copy(deg_sh.at[pl.ds(tb, _N - tb)],
                            deg_out0.at[pl.ds(tb, _N - tb)])

    @pl.when(cid == 1)
    def _out1():
        pltpu.sync_copy(agg_sh.at[pl.ds(rb, _CPR)], agg_out1.at[pl.ds(rb, _CPR)])
        pltpu.sync_copy(deg_sh.at[pl.ds(rb, _CPR)], deg_out1.at[pl.ds(rb, _CPR)])

        @pl.when(sid == 0)
        def _tail1():
            pltpu.sync_copy(agg_sh.at[pl.ds(tb, _N - tb)],
                            agg_out1.at[pl.ds(tb, _N - tb)])
            pltpu.sync_copy(deg_sh.at[pl.ds(tb, _N - tb)],
                            deg_out1.at[pl.ds(tb, _N - tb)])


_sc_agg = functools.partial(
    pl.kernel,
    mesh=plsc.VectorSubcoreMesh(core_axis_name="c", subcore_axis_name="s"),
    out_type=[
        jax.ShapeDtypeStruct((_N, _D), jnp.float32),
        jax.ShapeDtypeStruct((_N, _D), jnp.float32),
        jax.ShapeDtypeStruct((_N, _DEGW), jnp.float32),
        jax.ShapeDtypeStruct((_N, _DEGW), jnp.float32),
    ],
    scratch_types=[
        pltpu.VMEM((_NI, _C), jnp.int32),
        pltpu.VMEM((_NI, _C), jnp.int32),
        pltpu.VMEM((2, _C, _D), jnp.float32),
        pltpu.VMEM((_TAIL,), jnp.int32),
        pltpu.VMEM((_TAIL,), jnp.int32),
        pltpu.VMEM((_TAIL, _D), jnp.float32),
        pltpu.VMEM((_C, _DEGW), jnp.float32),
        pltpu.VMEM((_ZR, _DEGW), jnp.float32),
        pltpu.VMEM_SHARED((_N, _D), jnp.float32),
        pltpu.VMEM_SHARED((_N, _DEGW), jnp.float32),
        pltpu.SemaphoreType.DMA,
        pltpu.SemaphoreType.DMA,
        pltpu.SemaphoreType.DMA,
        pltpu.SemaphoreType.DMA,
        pltpu.SemaphoreType.DMA,
        pltpu.SemaphoreType.DMA,
        pltpu.SemaphoreType.DMA,
    ],
    compiler_params=pltpu.CompilerParams(use_tc_tiling_on_sc=False),
)(_sc_agg_body)


_BN = 1000


def _tc_body(p0, p1, d0, d1, w, b, o):
    deg = d0[:, 0:1] + d1[:, 0:1]
    norm = lax.rsqrt(deg)
    h = jnp.dot(p0[...] + p1[...], w[...], preferred_element_type=jnp.float32)
    o[...] = jnp.maximum(h * norm + b[...], 0.0)


def _tc_finish(p0, p1, d0, d1, w, b2d):
    return pl.pallas_call(
        _tc_body,
        grid=(_N // _BN,),
        in_specs=[
            pl.BlockSpec((_BN, _D), lambda i: (i, 0)),
            pl.BlockSpec((_BN, _D), lambda i: (i, 0)),
            pl.BlockSpec((_BN, _DEGW), lambda i: (i, 0)),
            pl.BlockSpec((_BN, _DEGW), lambda i: (i, 0)),
            pl.BlockSpec((_D, _D), lambda i: (0, 0)),
            pl.BlockSpec((1, _D), lambda i: (0, 0)),
        ],
        out_specs=pl.BlockSpec((_BN, _D), lambda i: (i, 0)),
        out_shape=jax.ShapeDtypeStruct((_N, _D), jnp.float32),
    )(p0, p1, d0, d1, w, b2d)


def kernel(x, edge_index, kernel, bias):
    src = edge_index[0]
    dst = edge_index[1]
    agg0, agg1, deg0, deg1 = _sc_agg(x, src, dst)
    return _tc_finish(agg0, agg1, deg0, deg1, kernel, bias.reshape(1, _D))


# speedup vs baseline: 13.3417x; 1.1066x over previous
import functools

import jax
import jax.numpy as jnp
from jax import lax
from jax.experimental import pallas as pl
from jax.experimental.pallas import tpu as pltpu
from jax.experimental.pallas import tpu_sc as plsc

_N = 10000
_E = 320000
_D = 128

_NC = 2
_NS = 16
_NW = _NC * _NS
_EPW = _E // _NW
_C = 96
_NFULL = _EPW // _C
_TAIL = _EPW - _NFULL * _C
_RPT = _N // _NS
_CPR = 624
_DEGW = 16
_ZRD = 25
_NR = 3
_NI = 4
_UN = 12
_NLOOP = _NFULL // _UN
_NEPI = _NFULL - _NLOOP * _UN


def _sc_agg_body(x_hbm, ei_hbm,
                 agg_out0, agg_out1, deg_out0, deg_out1,
                 idx_s, idx_d, rows, idx_st, idx_dt,
                 ones_v, zbuf_d, agg_sh, deg_sh,
                 isem0, isem1, isem2, isem3,
                 gsem0, gsem1, gsem2, ssem0, ssem1, ssem2, semt):
    isems = (isem0, isem1, isem2, isem3)
    gsems = (gsem0, gsem1, gsem2)
    ssems = (ssem0, ssem1, ssem2)
    cid = lax.axis_index("c")
    sid = lax.axis_index("s")
    wid = sid * _NC + cid
    ebase = wid * _EPW

    zeros16 = jnp.zeros((16,), jnp.float32)
    ones16 = jnp.ones((16,), jnp.float32)

    def fill_z(r, carry):
        for j in range(_D // 16):
            rows[0, r, pl.ds(j * 16, 16)] = zeros16
        return carry

    lax.fori_loop(0, _C, fill_z, 0)

    def fill_zd(r, carry):
        zbuf_d[r, :] = zeros16
        return carry

    lax.fori_loop(0, _ZRD, fill_zd, 0)

    def fill_o(r, carry):
        ones_v[r, :] = ones16
        return carry

    lax.fori_loop(0, _C, fill_o, 0)

    for t in range(6):
        pltpu.sync_copy(rows.at[0], agg_sh.at[pl.ds(sid * _RPT + t * _C, _C)])
    pltpu.sync_copy(rows.at[0, pl.ds(0, _RPT - 6 * _C)],
                    agg_sh.at[pl.ds(sid * _RPT + 6 * _C, _RPT - 6 * _C)])
    for t in range(_RPT // _ZRD):
        pltpu.sync_copy(zbuf_d, deg_sh.at[pl.ds(sid * _RPT + t * _ZRD, _ZRD)])
    plsc.subcore_barrier()

    def _idx_copies(g, r):
        off = pl.multiple_of(ebase + g * _C, 8)
        return (
            pltpu.make_async_copy(ei_hbm.at[0, pl.ds(off, _C)], idx_s.at[r],
                                  isems[r]),
            pltpu.make_async_copy(ei_hbm.at[1, pl.ds(off, _C)], idx_d.at[r],
                                  isems[r]),
        )

    def issue_idx(g, r):
        for c in _idx_copies(g, r):
            c.start()

    def wait_idx(g, r):
        for c in _idx_copies(g, r):
            c.wait()

    def _gather_copy(r, b):
        return pltpu.make_async_copy(
            x_hbm.at[idx_s.at[r]], rows.at[b], gsems[b])

    def _scatter_copy(r, b):
        return pltpu.make_async_copy(
            rows.at[b], agg_sh.at[idx_d.at[r]], ssems[b])

    issue_idx(0, 0)
    issue_idx(1, 1)
    issue_idx(2, 2)
    wait_idx(0, 0)
    _gather_copy(0, 0).start()
    wait_idx(1, 1)
    _gather_copy(1, 1).start()

    def chunk_step(h, i, r, guard):
        _gather_copy(r, i).wait()
        pltpu.async_copy(rows.at[i], agg_sh.at[idx_d.at[r]], ssems[i],
                         add=True)
        pltpu.sync_copy(ones_v, deg_sh.at[idx_s.at[r]], add=True)
        qi = (i + 2) % _NR
        qr = (r + 3) % _NI
        if guard:
            @pl.when(h >= 1)
            def _():
                _scatter_copy(qr, qi).wait()

            @pl.when(h + 3 < _NFULL)
            def _():
                issue_idx(h + 3, qr)

            @pl.when(h + 2 < _NFULL)
            def _():
                wait_idx(h + 2, (r + 2) % _NI)
                _gather_copy((r + 2) % _NI, qi).start()
        else:
            if h >= 1:
                _scatter_copy(qr, qi).wait()
            if h + 3 < _NFULL:
                issue_idx(h + 3, qr)
            if h + 2 < _NFULL:
                wait_idx(h + 2, (r + 2) % _NI)
                _gather_copy((r + 2) % _NI, qi).start()

    def body(t, carry):
        for u in range(_UN):
            h = _UN * t + u
            chunk_step(h, u % _NR, u % _NI, True)
        return carry

    lax.fori_loop(0, _NLOOP, body, 0)

    for h in range(_NLOOP * _UN, _NFULL):
        chunk_step(h, h % _NR, h % _NI, False)

    _scatter_copy((_NFULL - 1) % _NI, (_NFULL - 1) % _NR).wait()

    offt = pl.multiple_of(ebase + _NFULL * _C, 8)
    pltpu.sync_copy(ei_hbm.at[0, pl.ds(offt, _TAIL)], idx_st)
    pltpu.sync_copy(ei_hbm.at[1, pl.ds(offt, _TAIL)], idx_dt)
    pltpu.async_copy(x_hbm.at[idx_st], rows.at[0, pl.ds(0, _TAIL)], semt).wait()
    pltpu.sync_copy(rows.at[0, pl.ds(0, _TAIL)], agg_sh.at[idx_dt], add=True)
    pltpu.sync_copy(ones_v.at[pl.ds(0, _TAIL)], deg_sh.at[idx_st], add=True)

    plsc.subcore_barrier()

    rb = pl.multiple_of(sid * _CPR, 8)
    tb = _NS * _CPR

    @pl.when(cid == 0)
    def _out0():
        pltpu.sync_copy(agg_sh.at[pl.ds(rb, _CPR)], agg_out0.at[pl.ds(rb, _CPR)])
        pltpu.sync_copy(deg_sh.at[pl.ds(rb, _CPR)], deg_out0.at[pl.ds(rb, _CPR)])

        @pl.when(sid == 0)
        def _tail0():
            pltpu.sync_copy(agg_sh.at[pl.ds(tb, _N - tb)],
                            agg_out0.at[pl.ds(tb, _N - tb)])
            pltpu.sync_copy(deg_sh.at[pl.ds(tb, _N - tb)],
                            deg_out0.at[pl.ds(tb, _N - tb)])

    @pl.when(cid == 1)
    def _out1():
        pltpu.sync_copy(agg_sh.at[pl.ds(rb, _CPR)], agg_out1.at[pl.ds(rb, _CPR)])
        pltpu.sync_copy(deg_sh.at[pl.ds(rb, _CPR)], deg_out1.at[pl.ds(rb, _CPR)])

        @pl.when(sid == 0)
        def _tail1():
            pltpu.sync_copy(agg_sh.at[pl.ds(tb, _N - tb)],
                            agg_out1.at[pl.ds(tb, _N - tb)])
            pltpu.sync_copy(deg_sh.at[pl.ds(tb, _N - tb)],
                            deg_out1.at[pl.ds(tb, _N - tb)])


_sc_agg = functools.partial(
    pl.kernel,
    mesh=plsc.VectorSubcoreMesh(core_axis_name="c", subcore_axis_name="s"),
    out_type=[
        jax.ShapeDtypeStruct((_N, _D), jnp.float32),
        jax.ShapeDtypeStruct((_N, _D), jnp.float32),
        jax.ShapeDtypeStruct((_N, _DEGW), jnp.float32),
        jax.ShapeDtypeStruct((_N, _DEGW), jnp.float32),
    ],
    scratch_types=[
        pltpu.VMEM((_NI, _C), jnp.int32),
        pltpu.VMEM((_NI, _C), jnp.int32),
        pltpu.VMEM((_NR, _C, _D), jnp.float32),
        pltpu.VMEM((_TAIL,), jnp.int32),
        pltpu.VMEM((_TAIL,), jnp.int32),
        pltpu.VMEM((_C, _DEGW), jnp.float32),
        pltpu.VMEM((_ZRD, _DEGW), jnp.float32),
        pltpu.VMEM_SHARED((_N, _D), jnp.float32),
        pltpu.VMEM_SHARED((_N, _DEGW), jnp.float32),
        pltpu.SemaphoreType.DMA,
        pltpu.SemaphoreType.DMA,
        pltpu.SemaphoreType.DMA,
        pltpu.SemaphoreType.DMA,
        pltpu.SemaphoreType.DMA,
        pltpu.SemaphoreType.DMA,
        pltpu.SemaphoreType.DMA,
        pltpu.SemaphoreType.DMA,
        pltpu.SemaphoreType.DMA,
        pltpu.SemaphoreType.DMA,
        pltpu.SemaphoreType.DMA,
    ],
    compiler_params=pltpu.CompilerParams(use_tc_tiling_on_sc=False),
)(_sc_agg_body)


_BN = 1000


def _tc_body(p0, p1, d0, d1, w, b, o):
    deg = d0[:, 0:1] + d1[:, 0:1]
    norm = lax.rsqrt(deg)
    h = jnp.dot(p0[...] + p1[...], w[...], preferred_element_type=jnp.float32)
    o[...] = jnp.maximum(h * norm + b[...], 0.0)


def _tc_finish(p0, p1, d0, d1, w, b2d):
    return pl.pallas_call(
        _tc_body,
        grid=(_N // _BN,),
        in_specs=[
            pl.BlockSpec((_BN, _D), lambda i: (i, 0)),
            pl.BlockSpec((_BN, _D), lambda i: (i, 0)),
            pl.BlockSpec((_BN, _DEGW), lambda i: (i, 0)),
            pl.BlockSpec((_BN, _DEGW), lambda i: (i, 0)),
            pl.BlockSpec((_D, _D), lambda i: (0, 0)),
            pl.BlockSpec((1, _D), lambda i: (0, 0)),
        ],
        out_specs=pl.BlockSpec((_BN, _D), lambda i: (i, 0)),
        out_shape=jax.ShapeDtypeStruct((_N, _D), jnp.float32),
    )(p0, p1, d0, d1, w, b2d)


def kernel(x, edge_index, kernel, bias):
    agg0, agg1, deg0, deg1 = _sc_agg(x, edge_index)
    return _tc_finish(agg0, agg1, deg0, deg1, kernel, bias.reshape(1, _D))


# speedup vs baseline: 13.4170x; 1.0056x over previous
import functools

import jax
import jax.numpy as jnp
from jax import lax
from jax.experimental import pallas as pl
from jax.experimental.pallas import tpu as pltpu
from jax.experimental.pallas import tpu_sc as plsc

_N = 10000
_E = 320000
_D = 128

_NC = 2
_NS = 16
_NW = _NC * _NS
_EPW = _E // _NW
_C = 96
_NFULL = _EPW // _C
_TAIL = _EPW - _NFULL * _C
_RPT = _N // _NS
_CPR = 624
_DEGW = 16
_ZRD = 25
_NR = 3
_NI = 4
_UN = 12
_NLOOP = _NFULL // _UN
_NEPI = _NFULL - _NLOOP * _UN


def _sc_agg_body(x_hbm, ei_hbm,
                 agg_out0, agg_out1, deg_out0, deg_out1,
                 idx_s, idx_d, rows, idx_st, idx_dt,
                 ones_v, zbuf_d, agg_sh, deg_sh,
                 isem0, isem1, isem2, isem3,
                 gsem0, gsem1, gsem2, ssem0, ssem1, ssem2,
                 dsem0, dsem1, semt):
    isems = (isem0, isem1, isem2, isem3)
    gsems = (gsem0, gsem1, gsem2)
    ssems = (ssem0, ssem1, ssem2)
    dsems = (dsem0, dsem1)
    cid = lax.axis_index("c")
    sid = lax.axis_index("s")
    wid = sid * _NC + cid
    ebase = wid * _EPW

    zeros16 = jnp.zeros((16,), jnp.float32)
    ones16 = jnp.ones((16,), jnp.float32)

    def fill_z(r, carry):
        for j in range(_D // 16):
            rows[0, r, pl.ds(j * 16, 16)] = zeros16
        return carry

    lax.fori_loop(0, _C, fill_z, 0)

    def fill_zd(r, carry):
        zbuf_d[r, :] = zeros16
        return carry

    lax.fori_loop(0, _ZRD, fill_zd, 0)

    def fill_o(r, carry):
        ones_v[r, :] = ones16
        return carry

    lax.fori_loop(0, _C, fill_o, 0)

    for t in range(6):
        pltpu.sync_copy(rows.at[0], agg_sh.at[pl.ds(sid * _RPT + t * _C, _C)])
    pltpu.sync_copy(rows.at[0, pl.ds(0, _RPT - 6 * _C)],
                    agg_sh.at[pl.ds(sid * _RPT + 6 * _C, _RPT - 6 * _C)])
    for t in range(_RPT // _ZRD):
        pltpu.sync_copy(zbuf_d, deg_sh.at[pl.ds(sid * _RPT + t * _ZRD, _ZRD)])
    plsc.subcore_barrier()

    def _idx_copies(g, r):
        off = pl.multiple_of(ebase + g * _C, 8)
        return (
            pltpu.make_async_copy(ei_hbm.at[0, pl.ds(off, _C)], idx_s.at[r],
                                  isems[r]),
            pltpu.make_async_copy(ei_hbm.at[1, pl.ds(off, _C)], idx_d.at[r],
                                  isems[r]),
        )

    def issue_idx(g, r):
        for c in _idx_copies(g, r):
            c.start()

    def wait_idx(g, r):
        for c in _idx_copies(g, r):
            c.wait()

    def _gather_copy(r, b):
        return pltpu.make_async_copy(
            x_hbm.at[idx_s.at[r]], rows.at[b], gsems[b])

    def _scatter_copy(r, b):
        return pltpu.make_async_copy(
            rows.at[b], agg_sh.at[idx_d.at[r]], ssems[b])

    def _deg_copy(r, p):
        return pltpu.make_async_copy(
            ones_v, deg_sh.at[idx_s.at[r]], dsems[p])

    issue_idx(0, 0)
    issue_idx(1, 1)
    issue_idx(2, 2)
    wait_idx(0, 0)
    _gather_copy(0, 0).start()
    wait_idx(1, 1)
    _gather_copy(1, 1).start()

    def chunk_step(h, i, r, h2, guard):
        _gather_copy(r, i).wait()
        pltpu.async_copy(rows.at[i], agg_sh.at[idx_d.at[r]], ssems[i],
                         add=True)
        pltpu.async_copy(ones_v, deg_sh.at[idx_s.at[r]], dsems[h2],
                         add=True)
        qi = (i + 2) % _NR
        qr = (r + 3) % _NI
        if guard:
            @pl.when(h >= 1)
            def _():
                _scatter_copy(qr, qi).wait()
                _deg_copy(qr, 1 - h2).wait()

            @pl.when(h + 3 < _NFULL)
            def _():
                issue_idx(h + 3, qr)

            @pl.when(h + 2 < _NFULL)
            def _():
                wait_idx(h + 2, (r + 2) % _NI)
                _gather_copy((r + 2) % _NI, qi).start()
        else:
            if h >= 1:
                _scatter_copy(qr, qi).wait()
                _deg_copy(qr, 1 - h2).wait()
            if h + 3 < _NFULL:
                issue_idx(h + 3, qr)
            if h + 2 < _NFULL:
                wait_idx(h + 2, (r + 2) % _NI)
                _gather_copy((r + 2) % _NI, qi).start()

    def body(t, carry):
        for u in range(_UN):
            h = _UN * t + u
            chunk_step(h, u % _NR, u % _NI, u % 2, True)
        return carry

    lax.fori_loop(0, _NLOOP, body, 0)

    for h in range(_NLOOP * _UN, _NFULL):
        chunk_step(h, h % _NR, h % _NI, h % 2, False)

    _scatter_copy((_NFULL - 1) % _NI, (_NFULL - 1) % _NR).wait()
    _deg_copy((_NFULL - 1) % _NI, (_NFULL - 1) % 2).wait()

    offt = pl.multiple_of(ebase + _NFULL * _C, 8)
    pltpu.sync_copy(ei_hbm.at[0, pl.ds(offt, _TAIL)], idx_st)
    pltpu.sync_copy(ei_hbm.at[1, pl.ds(offt, _TAIL)], idx_dt)
    pltpu.async_copy(x_hbm.at[idx_st], rows.at[0, pl.ds(0, _TAIL)], semt).wait()
    pltpu.sync_copy(rows.at[0, pl.ds(0, _TAIL)], agg_sh.at[idx_dt], add=True)
    pltpu.sync_copy(ones_v.at[pl.ds(0, _TAIL)], deg_sh.at[idx_st], add=True)

    plsc.subcore_barrier()

    rb = pl.multiple_of(sid * _CPR, 8)
    tb = _NS * _CPR

    @pl.when(cid == 0)
    def _out0():
        pltpu.sync_copy(agg_sh.at[pl.ds(rb, _CPR)], agg_out0.at[pl.ds(rb, _CPR)])
        pltpu.sync_copy(deg_sh.at[pl.ds(rb, _CPR)], deg_out0.at[pl.ds(rb, _CPR)])

        @pl.when(sid == 0)
        def _tail0():
            pltpu.sync_copy(agg_sh.at[pl.ds(tb, _N - tb)],
                            agg_out0.at[pl.ds(tb, _N - tb)])
            pltpu.sync_copy(deg_sh.at[pl.ds(tb, _N - tb)],
                            deg_out0.at[pl.ds(tb, _N - tb)])

    @pl.when(cid == 1)
    def _out1():
        pltpu.sync_copy(agg_sh.at[pl.ds(rb, _CPR)], agg_out1.at[pl.ds(rb, _CPR)])
        pltpu.sync_copy(deg_sh.at[pl.ds(rb, _CPR)], deg_out1.at[pl.ds(rb, _CPR)])

        @pl.when(sid == 0)
        def _tail1():
            pltpu.sync_copy(agg_sh.at[pl.ds(tb, _N - tb)],
                            agg_out1.at[pl.ds(tb, _N - tb)])
            pltpu.sync_copy(deg_sh.at[pl.ds(tb, _N - tb)],
                            deg_out1.at[pl.ds(tb, _N - tb)])


_sc_agg = functools.partial(
    pl.kernel,
    mesh=plsc.VectorSubcoreMesh(core_axis_name="c", subcore_axis_name="s"),
    out_type=[
        jax.ShapeDtypeStruct((_N, _D), jnp.float32),
        jax.ShapeDtypeStruct((_N, _D), jnp.float32),
        jax.ShapeDtypeStruct((_N, _DEGW), jnp.float32),
        jax.ShapeDtypeStruct((_N, _DEGW), jnp.float32),
    ],
    scratch_types=[
        pltpu.VMEM((_NI, _C), jnp.int32),
        pltpu.VMEM((_NI, _C), jnp.int32),
        pltpu.VMEM((_NR, _C, _D), jnp.float32),
        pltpu.VMEM((_TAIL,), jnp.int32),
        pltpu.VMEM((_TAIL,), jnp.int32),
        pltpu.VMEM((_C, _DEGW), jnp.float32),
        pltpu.VMEM((_ZRD, _DEGW), jnp.float32),
        pltpu.VMEM_SHARED((_N, _D), jnp.float32),
        pltpu.VMEM_SHARED((_N, _DEGW), jnp.float32),
        pltpu.SemaphoreType.DMA,
        pltpu.SemaphoreType.DMA,
        pltpu.SemaphoreType.DMA,
        pltpu.SemaphoreType.DMA,
        pltpu.SemaphoreType.DMA,
        pltpu.SemaphoreType.DMA,
        pltpu.SemaphoreType.DMA,
        pltpu.SemaphoreType.DMA,
        pltpu.SemaphoreType.DMA,
        pltpu.SemaphoreType.DMA,
        pltpu.SemaphoreType.DMA,
        pltpu.SemaphoreType.DMA,
        pltpu.SemaphoreType.DMA,
    ],
    compiler_params=pltpu.CompilerParams(use_tc_tiling_on_sc=False),
)(_sc_agg_body)


_BN = 1000


def _tc_body(p0, p1, d0, d1, w, b, o):
    deg = d0[:, 0:1] + d1[:, 0:1]
    norm = lax.rsqrt(deg)
    h = jnp.dot(p0[...] + p1[...], w[...], preferred_element_type=jnp.float32)
    o[...] = jnp.maximum(h * norm + b[...], 0.0)


def _tc_finish(p0, p1, d0, d1, w, b2d):
    return pl.pallas_call(
        _tc_body,
        grid=(_N // _BN,),
        in_specs=[
            pl.BlockSpec((_BN, _D), lambda i: (i, 0)),
            pl.BlockSpec((_BN, _D), lambda i: (i, 0)),
            pl.BlockSpec((_BN, _DEGW), lambda i: (i, 0)),
            pl.BlockSpec((_BN, _DEGW), lambda i: (i, 0)),
            pl.BlockSpec((_D, _D), lambda i: (0, 0)),
            pl.BlockSpec((1, _D), lambda i: (0, 0)),
        ],
        out_specs=pl.BlockSpec((_BN, _D), lambda i: (i, 0)),
        out_shape=jax.ShapeDtypeStruct((_N, _D), jnp.float32),
    )(p0, p1, d0, d1, w, b2d)


def kernel(x, edge_index, kernel, bias):
    agg0, agg1, deg0, deg1 = _sc_agg(x, edge_index)
    return _tc_finish(agg0, agg1, deg0, deg1, kernel, bias.reshape(1, _D))
